# Initial kernel scaffold; baseline (speedup 1.0000x reference)
import jax
import jax.numpy as jnp
from jax.experimental import pallas as pl


def kernel(x, edge_index, edge_attr, smiles, batch, W1, b1, W2, b2, W3, b3, W4, b4):
    raise NotImplementedError("write your pallas kernel here")


# speedup vs baseline: 1.4687x; 1.4687x over previous
import functools
import jax
import jax.numpy as jnp
from jax.experimental import pallas as pl

N = 10000
E = 320000
D = 128
DE = 16
HID = 512
G = 256
DEPTH = 3

NPAD = 10240


def _pr_body(x_ref, w_ref, b_ref, p_ref, r_ref):
    y = jnp.dot(x_ref[...], w_ref[...], preferred_element_type=jnp.float32)
    p = jnp.exp(y + b_ref[...][None, :])
    p_ref[...] = p
    r_ref[...] = p / jnp.sum(p, axis=1, keepdims=True)


def _node_pr(x, Wx, b):
    din = x.shape[1]
    blk = 256
    grid = NPAD // blk
    return pl.pallas_call(
        _pr_body,
        grid=(grid,),
        in_specs=[
            pl.BlockSpec((blk, din), lambda i: (i, 0)),
            pl.BlockSpec((din, HID), lambda i: (0, 0)),
            pl.BlockSpec((HID,), lambda i: (0,)),
        ],
        out_specs=[
            pl.BlockSpec((blk, HID), lambda i: (i, 0)),
            pl.BlockSpec((blk, HID), lambda i: (i, 0)),
        ],
        out_shape=[
            jax.ShapeDtypeStruct((NPAD, HID), jnp.float32),
            jax.ShapeDtypeStruct((NPAD, HID), jnp.float32),
        ],
    )(x, Wx, b)


def _q_body(ea_ref, w_ref, q_ref):
    q_ref[...] = jnp.exp(
        jnp.dot(ea_ref[...], w_ref[...], preferred_element_type=jnp.float32))


def _edge_q(edge_attr, We):
    blk = 1280
    grid = E // blk
    return pl.pallas_call(
        _q_body,
        grid=(grid,),
        in_specs=[
            pl.BlockSpec((blk, DE), lambda i: (i, 0)),
            pl.BlockSpec((DE, HID), lambda i: (0, 0)),
        ],
        out_specs=pl.BlockSpec((blk, HID), lambda i: (i, 0)),
        out_shape=jax.ShapeDtypeStruct((E, HID), jnp.float32),
    )(edge_attr, We)


def _msg_body(pg_ref, q_ref, m_ref):
    u = pg_ref[...] * q_ref[...]
    m_ref[...] = u / jnp.sum(u, axis=1, keepdims=True)


def _edge_msg(Pg, Q):
    blk = 1280
    grid = E // blk
    return pl.pallas_call(
        _msg_body,
        grid=(grid,),
        in_specs=[
            pl.BlockSpec((blk, HID), lambda i: (i, 0)),
            pl.BlockSpec((blk, HID), lambda i: (i, 0)),
        ],
        out_specs=pl.BlockSpec((blk, HID), lambda i: (i, 0)),
        out_shape=jax.ShapeDtypeStruct((E, HID), jnp.float32),
    )(Pg, Q)


def _pool_body(b_ref, x_ref, acc_ref):
    i = pl.program_id(0)
    seg = b_ref[...]
    onehot = (seg == jax.lax.broadcasted_iota(jnp.int32, (G, seg.shape[1]), 0)
              ).astype(jnp.float32)
    part = jnp.dot(onehot, x_ref[...], preferred_element_type=jnp.float32)

    @pl.when(i == 0)
    def _init():
        acc_ref[...] = part

    @pl.when(i > 0)
    def _acc():
        acc_ref[...] += part


def _pool(batch_padded, x):
    blk = 1024
    grid = NPAD // blk
    return pl.pallas_call(
        _pool_body,
        grid=(grid,),
        in_specs=[
            pl.BlockSpec((1, blk), lambda i: (0, i)),
            pl.BlockSpec((blk, HID), lambda i: (i, 0)),
        ],
        out_specs=pl.BlockSpec((G, HID), lambda i: (0, 0)),
        out_shape=jax.ShapeDtypeStruct((G, HID), jnp.float32),
    )(batch_padded.reshape(1, NPAD), x)


def _readout_body(p_ref, w3_ref, b3_ref, w4_ref, b4_ref, o_ref):
    h = jnp.dot(p_ref[...], w3_ref[...], preferred_element_type=jnp.float32)
    h = h + b3_ref[...][None, :]
    o = jnp.dot(h, w4_ref[...], preferred_element_type=jnp.float32)
    o_ref[...] = o + b4_ref[...][None, :]


def _readout(pooled, W3, b3, W4, b4):
    return pl.pallas_call(
        _readout_body,
        in_specs=[pl.BlockSpec(pooled.shape, lambda: (0, 0)),
                  pl.BlockSpec(W3.shape, lambda: (0, 0)),
                  pl.BlockSpec(b3.shape, lambda: (0,)),
                  pl.BlockSpec(W4.shape, lambda: (0, 0)),
                  pl.BlockSpec(b4.shape, lambda: (0,))],
        out_specs=pl.BlockSpec((G, 1), lambda: (0, 0)),
        out_shape=jax.ShapeDtypeStruct((G, 1), jnp.float32),
    )(pooled, W3, b3, W4, b4)


def kernel(x, edge_index, edge_attr, smiles, batch, W1, b1, W2, b2, W3, b3, W4, b4):
    src, dst = edge_index[0], edge_index[1]
    Q1 = _edge_q(edge_attr, W1[D:])
    Q2 = _edge_q(edge_attr, W2[HID:])

    xp = jnp.zeros((NPAD, D), jnp.float32).at[:N].set(x)
    h = xp
    for layer in range(1 + DEPTH):
        if layer == 0:
            P, R = _node_pr(h, W1[:D], b1)
            Q = Q1
        else:
            P, R = _node_pr(h, W2[:HID], b2)
            Q = Q2
        Pg = jnp.take(P[:N], src, axis=0)
        msg = _edge_msg(Pg, Q)
        agg = jax.ops.segment_sum(msg, dst, num_segments=N)
        h = jnp.zeros((NPAD, HID), jnp.float32).at[:N].set(agg + R[:N])

    batch_padded = jnp.full((NPAD,), G, jnp.int32).at[:N].set(batch)
    pooled = _pool(batch_padded, h)
    return _readout(pooled, W3, b3, W4, b4)
